# baseline (speedup 1.0000x reference)
import jax
import jax.numpy as jnp
from jax.experimental import pallas as pl
from jax.experimental.pallas import tpu as pltpu

B, D_IN, H, N_LABELS = 4096, 5000, 1000, 8192
BM1 = 256
BM2 = 512
BN2 = 1024


def _l1_kernel(x_ref, w1_ref, b1_ref, y_ref, hid_ref):
    x = x_ref[...].astype(jnp.bfloat16)
    acc = jax.lax.dot_general(x, w1_ref[...], (((1,), (1,)), ((), ())),
                              preferred_element_type=jnp.float32)
    y = acc + b1_ref[...]
    y_ref[...] = y
    hid_ref[...] = jnp.maximum(y, 0.0).astype(jnp.bfloat16)


def _loss_kernel(hid_ref, w2_ref, b2_ref, t_ref, out_ref):
    n = pl.program_id(0)
    m = pl.program_id(1)
    z = jax.lax.dot_general(hid_ref[...], w2_ref[...], (((1,), (1,)), ((), ())),
                            preferred_element_type=jnp.float32)
    z = z + b2_ref[...]
    t = t_ref[...]
    e = jnp.maximum(z, 0.0) - z * t + jnp.log1p(jnp.exp(-jnp.abs(z)))
    s = jnp.sum(e)

    @pl.when((m == 0) & (n == 0))
    def _():
        out_ref[0, 0] = 0.0

    out_ref[0, 0] += s


def kernel(inputs, labels, W1, b1, W2, b2):
    x = inputs.astype(jnp.float32)
    w1b = W1.astype(jnp.bfloat16)
    w2b = W2.astype(jnp.bfloat16)
    b1r = b1.reshape(1, H)
    b2r = b2.reshape(1, N_LABELS)

    y, hid = pl.pallas_call(
        _l1_kernel,
        grid=(B // BM1,),
        in_specs=[
            pl.BlockSpec((BM1, D_IN), lambda i: (i, 0)),
            pl.BlockSpec((H, D_IN), lambda i: (0, 0)),
            pl.BlockSpec((1, H), lambda i: (0, 0)),
        ],
        out_specs=[
            pl.BlockSpec((BM1, H), lambda i: (i, 0)),
            pl.BlockSpec((BM1, H), lambda i: (i, 0)),
        ],
        out_shape=[
            jax.ShapeDtypeStruct((B, H), jnp.float32),
            jax.ShapeDtypeStruct((B, H), jnp.bfloat16),
        ],
    )(x, w1b, b1r)

    loss_sum = pl.pallas_call(
        _loss_kernel,
        grid=(N_LABELS // BN2, B // BM2),
        in_specs=[
            pl.BlockSpec((BM2, H), lambda n, m: (m, 0)),
            pl.BlockSpec((BN2, H), lambda n, m: (n, 0)),
            pl.BlockSpec((1, BN2), lambda n, m: (0, n)),
            pl.BlockSpec((BM2, BN2), lambda n, m: (m, n)),
        ],
        out_specs=pl.BlockSpec(memory_space=pltpu.SMEM),
        out_shape=jax.ShapeDtypeStruct((1, 1), jnp.float32),
    )(hid, w2b, b2r, labels)

    loss = loss_sum[0, 0] / (B * N_LABELS)
    return (y, loss)


# speedup vs baseline: 1.5562x; 1.5562x over previous
import jax
import jax.numpy as jnp
from jax.experimental import pallas as pl
from jax.experimental.pallas import tpu as pltpu

B, D_IN, H, N_LABELS = 4096, 5000, 1000, 8192
BM1 = 256
BM2 = 512
BN2 = 1024


def _l1_kernel(xt_ref, w1_ref, b1_ref, yt_ref, hidt_ref):
    xt = xt_ref[...].astype(jnp.bfloat16)
    acc = jax.lax.dot_general(w1_ref[...], xt, (((1,), (0,)), ((), ())),
                              preferred_element_type=jnp.float32)
    yt = acc + b1_ref[...]
    yt_ref[...] = yt
    hidt_ref[...] = jnp.maximum(yt, 0.0).astype(jnp.bfloat16)


def _loss_kernel(hidt_ref, w2t_ref, b2_ref, t_ref, out_ref):
    n = pl.program_id(0)
    m = pl.program_id(1)
    w2t = w2t_ref[...].astype(jnp.bfloat16)
    z = jax.lax.dot_general(hidt_ref[...], w2t, (((0,), (0,)), ((), ())),
                            preferred_element_type=jnp.float32)
    z = z + b2_ref[...]
    t = t_ref[...]
    e = jnp.maximum(z, 0.0) - z * t + jnp.log1p(jnp.exp(-jnp.abs(z)))
    s = jnp.sum(e)

    @pl.when((m == 0) & (n == 0))
    def _():
        out_ref[0, 0] = 0.0

    out_ref[0, 0] += s


def kernel(inputs, labels, W1, b1, W2, b2):
    xt = jnp.transpose(inputs.astype(jnp.float32))
    w2t = jnp.transpose(W2)
    w1b = W1.astype(jnp.bfloat16)
    b1c = b1.reshape(H, 1)
    b2r = b2.reshape(1, N_LABELS)

    yt, hidt = pl.pallas_call(
        _l1_kernel,
        grid=(B // BM1,),
        in_specs=[
            pl.BlockSpec((D_IN, BM1), lambda i: (0, i)),
            pl.BlockSpec((H, D_IN), lambda i: (0, 0)),
            pl.BlockSpec((H, 1), lambda i: (0, 0)),
        ],
        out_specs=[
            pl.BlockSpec((H, BM1), lambda i: (0, i)),
            pl.BlockSpec((H, BM1), lambda i: (0, i)),
        ],
        out_shape=[
            jax.ShapeDtypeStruct((H, B), jnp.float32),
            jax.ShapeDtypeStruct((H, B), jnp.bfloat16),
        ],
    )(xt, w1b, b1c)

    loss_sum = pl.pallas_call(
        _loss_kernel,
        grid=(N_LABELS // BN2, B // BM2),
        in_specs=[
            pl.BlockSpec((H, BM2), lambda n, m: (0, m)),
            pl.BlockSpec((H, BN2), lambda n, m: (0, n)),
            pl.BlockSpec((1, BN2), lambda n, m: (0, n)),
            pl.BlockSpec((BM2, BN2), lambda n, m: (m, n)),
        ],
        out_specs=pl.BlockSpec(memory_space=pltpu.SMEM),
        out_shape=jax.ShapeDtypeStruct((1, 1), jnp.float32),
    )(hidt, w2t, b2r, labels)

    loss = loss_sum[0, 0] / (B * N_LABELS)
    return (jnp.transpose(yt), loss)


# speedup vs baseline: 1.7080x; 1.0976x over previous
import jax
import jax.numpy as jnp
from jax.experimental import pallas as pl
from jax.experimental.pallas import tpu as pltpu

B, D_IN, H, N_LABELS = 4096, 5000, 1000, 8192
BM1 = 256
BM2 = 512
BN2 = 1024


def _l1_kernel(xt_ref, w1_ref, b1_ref, yt_ref, hidt_ref):
    xt = xt_ref[...].astype(jnp.bfloat16)
    acc = jax.lax.dot_general(w1_ref[...], xt, (((1,), (0,)), ((), ())),
                              preferred_element_type=jnp.float32)
    yt = acc + b1_ref[...]
    yt_ref[...] = yt
    hidt_ref[...] = jnp.maximum(yt, 0.0).astype(jnp.bfloat16)


def _loss_kernel(hidt_ref, w2t_ref, b2_ref, t_ref, out_ref):
    n = pl.program_id(0)
    m = pl.program_id(1)
    w2t = w2t_ref[...].astype(jnp.bfloat16)
    z = jax.lax.dot_general(hidt_ref[...], w2t, (((0,), (0,)), ((), ())),
                            preferred_element_type=jnp.float32)
    zb = z.astype(jnp.bfloat16) + b2_ref[...]
    tb = t_ref[...].astype(jnp.bfloat16)
    e = (jnp.maximum(zb, 0.0) - zb * tb
         + jnp.log1p(jnp.exp(-jnp.abs(zb))))
    s = jnp.sum(e.astype(jnp.float32))

    @pl.when((m == 0) & (n == 0))
    def _():
        out_ref[0, 0] = 0.0

    out_ref[0, 0] += s


def kernel(inputs, labels, W1, b1, W2, b2):
    xt = jnp.transpose(inputs.astype(jnp.float32))
    w2t = jnp.transpose(W2)
    w1b = W1.astype(jnp.bfloat16)
    b1c = b1.reshape(H, 1)
    b2r = b2.reshape(1, N_LABELS).astype(jnp.bfloat16)

    yt, hidt = pl.pallas_call(
        _l1_kernel,
        grid=(B // BM1,),
        in_specs=[
            pl.BlockSpec((D_IN, BM1), lambda i: (0, i)),
            pl.BlockSpec((H, D_IN), lambda i: (0, 0)),
            pl.BlockSpec((H, 1), lambda i: (0, 0)),
        ],
        out_specs=[
            pl.BlockSpec((H, BM1), lambda i: (0, i)),
            pl.BlockSpec((H, BM1), lambda i: (0, i)),
        ],
        out_shape=[
            jax.ShapeDtypeStruct((H, B), jnp.float32),
            jax.ShapeDtypeStruct((H, B), jnp.bfloat16),
        ],
    )(xt, w1b, b1c)

    loss_sum = pl.pallas_call(
        _loss_kernel,
        grid=(N_LABELS // BN2, B // BM2),
        in_specs=[
            pl.BlockSpec((H, BM2), lambda n, m: (0, m)),
            pl.BlockSpec((H, BN2), lambda n, m: (0, n)),
            pl.BlockSpec((1, BN2), lambda n, m: (0, n)),
            pl.BlockSpec((BM2, BN2), lambda n, m: (m, n)),
        ],
        out_specs=pl.BlockSpec(memory_space=pltpu.SMEM),
        out_shape=jax.ShapeDtypeStruct((1, 1), jnp.float32),
    )(hidt, w2t, b2r, labels)

    loss = loss_sum[0, 0] / (B * N_LABELS)
    return (jnp.transpose(yt), loss)


# speedup vs baseline: 1.8129x; 1.0614x over previous
import jax
import jax.numpy as jnp
from jax.experimental import pallas as pl
from jax.experimental.pallas import tpu as pltpu

B, D_IN, H, N_LABELS = 4096, 5000, 1000, 8192
BM1 = 256
BM2 = 512
BN2 = 1024
M2 = B // BM2
N2 = N_LABELS // BN2
NTILES = M2 * N2
NSTEPS = NTILES // 2 + 1


def _l1_kernel(xt_ref, w1_ref, b1_ref, yt_ref, hidt_ref):
    xt = xt_ref[...].astype(jnp.bfloat16)
    acc = jax.lax.dot_general(w1_ref[...], xt, (((1,), (0,)), ((), ())),
                              preferred_element_type=jnp.float32)
    yt = acc + b1_ref[...]
    yt_ref[...] = yt
    hidt_ref[...] = jnp.maximum(yt, 0.0).astype(jnp.bfloat16)


def _bce_tile(zb, b2b, t):
    z = zb + b2b
    tb = t.astype(jnp.bfloat16)
    e = jnp.maximum(z, 0.0) - z * tb + jnp.log1p(jnp.exp(-jnp.abs(z)))
    return jnp.sum(e.astype(jnp.float32))


def _loss_kernel(hidA_ref, w2A_ref, hidB_ref, w2B_ref,
                 tA_ref, b2A_ref, tB_ref, b2B_ref,
                 out_ref, zA_ref, zB_ref):
    s = pl.program_id(0)

    sA = _bce_tile(zA_ref[...], b2A_ref[...], tA_ref[...])
    sB = _bce_tile(zB_ref[...], b2B_ref[...], tB_ref[...])

    w2A = w2A_ref[...].astype(jnp.bfloat16)
    zA = jax.lax.dot_general(hidA_ref[...], w2A, (((0,), (0,)), ((), ())),
                             preferred_element_type=jnp.float32)
    zA_ref[...] = zA.astype(jnp.bfloat16)
    w2B = w2B_ref[...].astype(jnp.bfloat16)
    zB = jax.lax.dot_general(hidB_ref[...], w2B, (((0,), (0,)), ((), ())),
                             preferred_element_type=jnp.float32)
    zB_ref[...] = zB.astype(jnp.bfloat16)

    @pl.when(s == 0)
    def _():
        out_ref[0, 0] = 0.0

    @pl.when(s > 0)
    def _():
        out_ref[0, 0] += sA + sB


def _q_nm(q):
    q = jnp.clip(q, 0, NTILES - 1)
    return q // M2, q % M2


def kernel(inputs, labels, W1, b1, W2, b2):
    xt = jnp.transpose(inputs.astype(jnp.float32))
    w2t = jnp.transpose(W2)
    w1b = W1.astype(jnp.bfloat16)
    b1c = b1.reshape(H, 1)
    b2r = b2.reshape(1, N_LABELS).astype(jnp.bfloat16)

    yt, hidt = pl.pallas_call(
        _l1_kernel,
        grid=(B // BM1,),
        in_specs=[
            pl.BlockSpec((D_IN, BM1), lambda i: (0, i)),
            pl.BlockSpec((H, D_IN), lambda i: (0, 0)),
            pl.BlockSpec((H, 1), lambda i: (0, 0)),
        ],
        out_specs=[
            pl.BlockSpec((H, BM1), lambda i: (0, i)),
            pl.BlockSpec((H, BM1), lambda i: (0, i)),
        ],
        out_shape=[
            jax.ShapeDtypeStruct((H, B), jnp.float32),
            jax.ShapeDtypeStruct((H, B), jnp.bfloat16),
        ],
    )(xt, w1b, b1c)

    def hid_spec(off):
        return pl.BlockSpec(
            (H, BM2), lambda s: (0, _q_nm(2 * s + off)[1]))

    def w2_spec(off):
        return pl.BlockSpec(
            (H, BN2), lambda s: (_q_nm(2 * s + off)[0] * 0, _q_nm(2 * s + off)[0]))

    def t_spec(off):
        return pl.BlockSpec(
            (BM2, BN2),
            lambda s: (_q_nm(2 * s + off)[1], _q_nm(2 * s + off)[0]))

    def b2_spec(off):
        return pl.BlockSpec(
            (1, BN2), lambda s: (0, _q_nm(2 * s + off)[0]))

    loss_sum = pl.pallas_call(
        _loss_kernel,
        grid=(NSTEPS,),
        in_specs=[
            hid_spec(0), w2_spec(0),
            hid_spec(1), w2_spec(1),
            t_spec(-2), b2_spec(-2),
            t_spec(-1), b2_spec(-1),
        ],
        out_specs=pl.BlockSpec(memory_space=pltpu.SMEM),
        out_shape=jax.ShapeDtypeStruct((1, 1), jnp.float32),
        scratch_shapes=[
            pltpu.VMEM((BM2, BN2), jnp.bfloat16),
            pltpu.VMEM((BM2, BN2), jnp.bfloat16),
        ],
    )(hidt, w2t, hidt, w2t, labels, b2r, labels, b2r)

    loss = loss_sum[0, 0] / (B * N_LABELS)
    return (jnp.transpose(yt), loss)
